# baseline (speedup 1.0000x reference)
import functools

import jax
import jax.numpy as jnp
from jax import lax
from jax.experimental import pallas as pl
from jax.experimental.pallas import tpu as pltpu
from jax.experimental.pallas import tpu_sc as plsc


def _make_lookup(L: int, B: int, V: int, D: int):
    info = plsc.get_sparse_core_info()
    NC, NS, NL = info.num_cores, info.num_subcores, info.num_lanes
    NW = NC * NS
    W = 128

    assert B == NW * W and NL == 16 and D % NL == 0

    mesh = plsc.VectorSubcoreMesh(core_axis_name="c", subcore_axis_name="s")

    @functools.partial(
        pl.kernel,
        mesh=mesh,
        out_type=jax.ShapeDtypeStruct((L, D, B), jnp.float32),
        scratch_types=[
            pltpu.VMEM((L, W), jnp.int32),
            pltpu.VMEM((2, W, D), jnp.float32),
            pltpu.VMEM((2, D, W), jnp.float32),
            pltpu.SemaphoreType.DMA((2,)),
            pltpu.SemaphoreType.DMA((2,)),
        ],
        compiler_params=pltpu.CompilerParams(
            use_tc_tiling_on_sc=False, needs_layout_passes=False
        ),
    )
    def lookup_kernel(table_hbm, idx_hbm, out_hbm, idx_v, gbuf, tbuf, gsem, ssem):
        wid = lax.axis_index("s") * NC + lax.axis_index("c")
        col = wid * W

        pltpu.sync_copy(idx_hbm.at[:, pl.ds(col, W)], idx_v)

        def start_gather(l, b):
            pltpu.async_copy(
                table_hbm.at[idx_v.at[l]], gbuf.at[b], gsem.at[b]
            )

        def wait_gather(b):
            pltpu.make_async_copy(
                table_hbm.at[pl.ds(0, W)], gbuf.at[b], gsem.at[b]
            ).wait()

        def start_store(l, b):
            pltpu.async_copy(
                tbuf.at[b], out_hbm.at[l, :, pl.ds(col, W)], ssem.at[b]
            )

        def wait_store(l, b):
            pltpu.make_async_copy(
                tbuf.at[b], out_hbm.at[l, :, pl.ds(col, W)], ssem.at[b]
            ).wait()

        lanes = lax.iota(jnp.int32, NL)
        row_ids = [lanes + j * NL for j in range(W // NL)]

        def transpose(b):
            src = gbuf.at[b]
            dst = tbuf.at[b]
            for d in range(D):
                cols = jnp.full((NL,), d, jnp.int32)
                for j in range(W // NL):
                    vec = plsc.load_gather(src, [row_ids[j], cols])
                    dst[d, pl.ds(j * NL, NL)] = vec

        start_gather(0, 0)

        def group(g, carry):
            for b in range(2):
                l = g * 2 + b

                @pl.when(l + 1 < L)
                def _():
                    start_gather(l + 1, 1 - b)

                wait_gather(b)

                @pl.when(l >= 2)
                def _():
                    wait_store(l - 2, b)

                transpose(b)
                start_store(l, b)
            return carry

        lax.fori_loop(0, L // 2, group, 0)
        wait_store(L - 2, 0)
        wait_store(L - 1, 1)

    return lookup_kernel


def kernel(idx, table):
    B, L = idx.shape
    V, D = table.shape
    idx_t = idx.T
    out_t = _make_lookup(L, B, V, D)(table, idx_t)
    return jnp.transpose(out_t, (2, 0, 1))


# speedup vs baseline: 1.6400x; 1.6400x over previous
import functools

import jax
import jax.numpy as jnp
from jax import lax
from jax.experimental import pallas as pl
from jax.experimental.pallas import tpu as pltpu
from jax.experimental.pallas import tpu_sc as plsc


def _make_lookup(L: int, B: int, V: int, D: int):
    info = plsc.get_sparse_core_info()
    NC, NS, NL = info.num_cores, info.num_subcores, info.num_lanes
    NW = NC * NS
    W = 128

    assert B == NW * W and NL == 16 and D % NL == 0

    mesh = plsc.VectorSubcoreMesh(core_axis_name="c", subcore_axis_name="s")

    @functools.partial(
        pl.kernel,
        mesh=mesh,
        out_type=jax.ShapeDtypeStruct((L, D, B), jnp.float32),
        scratch_types=[
            pltpu.VMEM((L, W), jnp.int32),
            pltpu.VMEM((2, W, D), jnp.float32),
            pltpu.VMEM((2, D, W + 8), jnp.float32),
            pltpu.SemaphoreType.DMA((2,)),
            pltpu.SemaphoreType.DMA((2,)),
        ],
        compiler_params=pltpu.CompilerParams(
            use_tc_tiling_on_sc=False,
            needs_layout_passes=False,
            disable_bounds_checks=True,
        ),
    )
    def lookup_kernel(table_hbm, idx_hbm, out_hbm, idx_v, gbuf, tbuf, gsem, ssem):
        wid = lax.axis_index("s") * NC + lax.axis_index("c")
        col = wid * W

        pltpu.sync_copy(idx_hbm.at[:, pl.ds(col, W)], idx_v)

        def start_gather(l, b):
            pltpu.async_copy(
                table_hbm.at[idx_v.at[l]], gbuf.at[b], gsem.at[b]
            )

        def wait_gather(b):
            pltpu.make_async_copy(
                table_hbm.at[pl.ds(0, W)], gbuf.at[b], gsem.at[b]
            ).wait()

        def start_store(l, b):
            pltpu.async_copy(
                tbuf.at[b].at[:, pl.ds(0, W)],
                out_hbm.at[l, :, pl.ds(col, W)],
                ssem.at[b],
            )

        def wait_store(l, b):
            pltpu.make_async_copy(
                tbuf.at[b].at[:, pl.ds(0, W)],
                out_hbm.at[l, :, pl.ds(col, W)],
                ssem.at[b],
            ).wait()

        lanes = lax.iota(jnp.int32, NL)
        dg_rows = [lanes + g * NL for g in range(D // NL)]

        def transpose(b):
            src = gbuf.at[b]
            dst = tbuf.at[b]
            for t in range(W):
                cols = jnp.full((NL,), t, jnp.int32)
                for g in range(D // NL):
                    vec = src[t, pl.ds(g * NL, NL)]
                    plsc.store_scatter(dst, [dg_rows[g], cols], vec)

        start_gather(0, 0)

        def group(g, carry):
            for b in range(2):
                l = g * 2 + b

                @pl.when(l + 1 < L)
                def _():
                    start_gather(l + 1, 1 - b)

                wait_gather(b)

                @pl.when(l >= 2)
                def _():
                    wait_store(l - 2, b)

                transpose(b)
                start_store(l, b)
            return carry

        lax.fori_loop(0, L // 2, group, 0)
        wait_store(L - 2, 0)
        wait_store(L - 1, 1)

    return lookup_kernel


def kernel(idx, table):
    B, L = idx.shape
    V, D = table.shape
    idx_t = idx.T
    out_t = _make_lookup(L, B, V, D)(table, idx_t)
    return jnp.transpose(out_t, (2, 0, 1))


# speedup vs baseline: 2.2059x; 1.3451x over previous
import functools

import jax
import jax.numpy as jnp
from jax import lax
from jax.experimental import pallas as pl
from jax.experimental.pallas import tpu as pltpu
from jax.experimental.pallas import tpu_sc as plsc


def _make_lookup(L: int, B: int, V: int, D: int):
    info = plsc.get_sparse_core_info()
    NC, NS, NL = info.num_cores, info.num_subcores, info.num_lanes
    NW = NC * NS
    W = 128

    assert B == NW * W and NL == 16 and D % NL == 0

    mesh = plsc.VectorSubcoreMesh(core_axis_name="c", subcore_axis_name="s")

    @functools.partial(
        pl.kernel,
        mesh=mesh,
        out_type=jax.ShapeDtypeStruct((L, D, B), jnp.float32),
        scratch_types=[
            pltpu.VMEM((L, W), jnp.int32),
            pltpu.VMEM((2, W, D), jnp.float32),
            pltpu.VMEM((2, D, W + 8), jnp.float32),
            pltpu.SemaphoreType.DMA((2,)),
            pltpu.SemaphoreType.DMA((2,)),
        ],
        compiler_params=pltpu.CompilerParams(
            use_tc_tiling_on_sc=False,
            needs_layout_passes=False,
            disable_bounds_checks=True,
        ),
    )
    def lookup_kernel(table_hbm, idx_hbm, out_hbm, idx_v, gbuf, tbuf, gsem, ssem):
        wid = lax.axis_index("s") * NC + lax.axis_index("c")
        col = wid * W

        pltpu.sync_copy(idx_hbm.at[:, pl.ds(col, W)], idx_v)

        def start_gather(l, b):
            pltpu.async_copy(
                table_hbm.at[idx_v.at[l]], gbuf.at[b], gsem.at[b]
            )

        def wait_gather(b):
            pltpu.make_async_copy(
                table_hbm.at[pl.ds(0, W)], gbuf.at[b], gsem.at[b]
            ).wait()

        def start_store(l, b):
            pltpu.async_copy(
                tbuf.at[b].at[:, pl.ds(0, W)],
                out_hbm.at[l, :, pl.ds(col, W)],
                ssem.at[b],
            )

        def wait_store(l, b):
            pltpu.make_async_copy(
                tbuf.at[b].at[:, pl.ds(0, W)],
                out_hbm.at[l, :, pl.ds(col, W)],
                ssem.at[b],
            ).wait()

        lanes = lax.iota(jnp.int32, NL)
        dg_rows = [lanes + g * NL for g in range(D // NL)]

        def transpose(b):
            src = gbuf.at[b]
            dst = tbuf.at[b]

            @plsc.parallel_loop(0, W, unroll=4)
            def t_body(t):
                cols = jnp.full((NL,), 0, jnp.int32) + t
                for g in range(D // NL):
                    vec = src[t, pl.ds(g * NL, NL)]
                    plsc.store_scatter(dst, [dg_rows[g], cols], vec)

        start_gather(0, 0)

        def group(g, carry):
            for b in range(2):
                l = g * 2 + b

                @pl.when(l + 1 < L)
                def _():
                    start_gather(l + 1, 1 - b)

                wait_gather(b)

                @pl.when(l >= 2)
                def _():
                    wait_store(l - 2, b)

                transpose(b)
                start_store(l, b)
            return carry

        lax.fori_loop(0, L // 2, group, 0)
        wait_store(L - 2, 0)
        wait_store(L - 1, 1)

    return lookup_kernel


def kernel(idx, table):
    B, L = idx.shape
    V, D = table.shape
    idx_t = idx.T
    out_t = _make_lookup(L, B, V, D)(table, idx_t)
    return jnp.transpose(out_t, (2, 0, 1))


# speedup vs baseline: 2.2064x; 1.0002x over previous
import functools

import jax
import jax.numpy as jnp
from jax import lax
from jax.experimental import pallas as pl
from jax.experimental.pallas import tpu as pltpu
from jax.experimental.pallas import tpu_sc as plsc


def _make_lookup(L: int, B: int, V: int, D: int):
    info = plsc.get_sparse_core_info()
    NC, NS, NL = info.num_cores, info.num_subcores, info.num_lanes
    NW = NC * NS
    W = 128

    assert B == NW * W and NL == 16 and D % NL == 0

    mesh = plsc.VectorSubcoreMesh(core_axis_name="c", subcore_axis_name="s")

    @functools.partial(
        pl.kernel,
        mesh=mesh,
        out_type=jax.ShapeDtypeStruct((L, D, B), jnp.float32),
        scratch_types=[
            pltpu.VMEM((L, W), jnp.int32),
            pltpu.VMEM((2, W, D), jnp.float32),
            pltpu.VMEM((2, D, W + 8), jnp.float32),
            pltpu.SemaphoreType.DMA((2,)),
            pltpu.SemaphoreType.DMA((2,)),
        ],
        compiler_params=pltpu.CompilerParams(
            use_tc_tiling_on_sc=False,
            needs_layout_passes=False,
            disable_bounds_checks=True,
        ),
    )
    def lookup_kernel(table_hbm, idx_hbm, out_hbm, idx_v, gbuf, tbuf, gsem, ssem):
        wid = lax.axis_index("s") * NC + lax.axis_index("c")
        col = wid * W

        pltpu.sync_copy(idx_hbm.at[:, pl.ds(col, W)], idx_v)

        def start_gather(l, b):
            pltpu.async_copy(
                table_hbm.at[idx_v.at[l]], gbuf.at[b], gsem.at[b]
            )

        def wait_gather(b):
            pltpu.make_async_copy(
                table_hbm.at[pl.ds(0, W)], gbuf.at[b], gsem.at[b]
            ).wait()

        def start_store(l, b):
            pltpu.async_copy(
                tbuf.at[b].at[:, pl.ds(0, W)],
                out_hbm.at[l, :, pl.ds(col, W)],
                ssem.at[b],
            )

        def wait_store(l, b):
            pltpu.make_async_copy(
                tbuf.at[b].at[:, pl.ds(0, W)],
                out_hbm.at[l, :, pl.ds(col, W)],
                ssem.at[b],
            ).wait()

        lanes = lax.iota(jnp.int32, NL)
        dg_rows = [lanes + g * NL for g in range(D // NL)]

        def transpose(b):
            src = gbuf.at[b]
            dst = tbuf.at[b]

            @plsc.parallel_loop(0, W, unroll=8)
            def t_body(t):
                cols = jnp.full((NL,), 0, jnp.int32) + t
                for g in range(D // NL):
                    vec = src[t, pl.ds(g * NL, NL)]
                    plsc.store_scatter(dst, [dg_rows[g], cols], vec)

        start_gather(0, 0)

        def group(g, carry):
            for b in range(2):
                l = g * 2 + b

                @pl.when(l + 1 < L)
                def _():
                    start_gather(l + 1, 1 - b)

                wait_gather(b)

                @pl.when(l >= 2)
                def _():
                    wait_store(l - 2, b)

                transpose(b)
                start_store(l, b)
            return carry

        lax.fori_loop(0, L // 2, group, 0)
        wait_store(L - 2, 0)
        wait_store(L - 1, 1)

    return lookup_kernel


def kernel(idx, table):
    B, L = idx.shape
    V, D = table.shape
    idx_t = idx.T
    out_t = _make_lookup(L, B, V, D)(table, idx_t)
    return jnp.transpose(out_t, (2, 0, 1))
